# Initial kernel scaffold; baseline (speedup 1.0000x reference)
import jax
import jax.numpy as jnp
from jax.experimental import pallas as pl


def kernel(x, t_table, c_table, W, b):
    raise NotImplementedError("write your pallas kernel here")


# speedup vs baseline: 9.2301x; 9.2301x over previous
import functools

import jax
import jax.numpy as jnp
from jax.experimental import pallas as pl
from jax.experimental.pallas import tpu as pltpu

_CHANS = 64
_SAMPLES = 1024
_DIM = 128
_SB = 128


def _body(xt_ref, t_ref, c_ref, w_ref, b_ref, out_ref):
    xv = xt_ref[0]
    w = w_ref[0]
    base = (t_ref[:][:, None, :]
            + (c_ref[:] + b_ref[:])[None, :, :])
    res = xv[:, :, None] * w[None, None, :] + base
    out_ref[0] = res.reshape(_SB * _CHANS, _DIM)


@functools.partial(jax.jit, static_argnames=())
def kernel(x, t_table, c_table, W, b):
    batch = x.shape[0]
    xt = jnp.transpose(x[:, 0], (0, 2, 1))
    wv = W[:, 0][None, :]
    bv = b[None, :]
    n_sb = _SAMPLES // _SB
    grid = (batch, n_sb)
    return pl.pallas_call(
        _body,
        grid=grid,
        in_specs=[
            pl.BlockSpec((1, _SB, _CHANS), lambda bi, si: (bi, si, 0)),
            pl.BlockSpec((_SB, _DIM), lambda bi, si: (si, 0)),
            pl.BlockSpec((_CHANS, _DIM), lambda bi, si: (0, 0)),
            pl.BlockSpec((1, _DIM), lambda bi, si: (0, 0)),
            pl.BlockSpec((1, _DIM), lambda bi, si: (0, 0)),
        ],
        out_specs=pl.BlockSpec((1, _SB * _CHANS, _DIM), lambda bi, si: (bi, si, 0)),
        out_shape=jax.ShapeDtypeStruct((batch, _SAMPLES * _CHANS, _DIM), jnp.float32),
        compiler_params=pltpu.CompilerParams(
            dimension_semantics=("parallel", "parallel"),
        ),
    )(xt, t_table, c_table, wv, bv)


# speedup vs baseline: 10.4102x; 1.1279x over previous
import functools

import jax
import jax.numpy as jnp
from jax.experimental import pallas as pl
from jax.experimental.pallas import tpu as pltpu

_CHANS = 64
_SAMPLES = 1024
_DIM = 128
_SB = 256


def _body(xt_ref, t_ref, c_ref, w_ref, b_ref, out_ref):
    xv = xt_ref[0]
    w = w_ref[0]
    base = (t_ref[:][:, None, :]
            + (c_ref[:] + b_ref[:])[None, :, :])
    res = xv[:, :, None] * w[None, None, :] + base
    out_ref[0] = res.reshape(_SB * _CHANS, _DIM)


@functools.partial(jax.jit, static_argnames=())
def kernel(x, t_table, c_table, W, b):
    batch = x.shape[0]
    xt = jnp.transpose(x[:, 0], (0, 2, 1))
    wv = W[:, 0][None, :]
    bv = b[None, :]
    n_sb = _SAMPLES // _SB
    grid = (batch, n_sb)
    return pl.pallas_call(
        _body,
        grid=grid,
        in_specs=[
            pl.BlockSpec((1, _SB, _CHANS), lambda bi, si: (bi, si, 0)),
            pl.BlockSpec((_SB, _DIM), lambda bi, si: (si, 0)),
            pl.BlockSpec((_CHANS, _DIM), lambda bi, si: (0, 0)),
            pl.BlockSpec((1, _DIM), lambda bi, si: (0, 0)),
            pl.BlockSpec((1, _DIM), lambda bi, si: (0, 0)),
        ],
        out_specs=pl.BlockSpec((1, _SB * _CHANS, _DIM), lambda bi, si: (bi, si, 0)),
        out_shape=jax.ShapeDtypeStruct((batch, _SAMPLES * _CHANS, _DIM), jnp.float32),
        compiler_params=pltpu.CompilerParams(
            dimension_semantics=("parallel", "parallel"),
        ),
    )(xt, t_table, c_table, wv, bv)
